# Initial kernel scaffold; baseline (speedup 1.0000x reference)
import jax
import jax.numpy as jnp
from jax.experimental import pallas as pl


def kernel(node_attr, edge_index, edge_attr, edge_sh, W_fc1, b_fc1, W_fc2, b_fc2):
    raise NotImplementedError("write your pallas kernel here")


# speedup vs baseline: 1.2776x; 1.2776x over previous
import functools

import jax
import jax.numpy as jnp
import numpy as np
from jax import lax
from jax.experimental import pallas as pl
from jax.experimental.pallas import tpu as pltpu
from jax.experimental.pallas import tpu_sc as plsc

NS = 16
NV = 4
NFEAT = 48
W0N = NS * NS
W1N = NS * NV
WPAD = 512
NORM = 1.0 / np.sqrt(float(NS))

NC = 2
NSUB = 16
NW = NC * NSUB

BATCH = 125
KSTR = 8
CHUNK = BATCH * KSTR

TPW = 32


def _sc_gather_body(nchunks, node_hbm, idx_hbm, out_hbm, idx_v, rows_v, sem):
  c = lax.axis_index("c")
  s = lax.axis_index("s")
  wid = s * NC + c

  def chunk(j, carry):
    srow = (wid * nchunks + j) * KSTR
    pltpu.sync_copy(idx_hbm.at[pl.ds(srow, KSTR)], idx_v)
    cps = [
        pltpu.async_copy(node_hbm.at[idx_v.at[jj]], rows_v.at[jj], sem)
        for jj in range(KSTR)
    ]
    for cp in cps:
      cp.wait()
    pltpu.sync_copy(rows_v, out_hbm.at[pl.ds(srow, KSTR)])
    return carry

  lax.fori_loop(0, nchunks, chunk, 0)


def _sc_gather(node_attr, dst2d):
  nstreams = dst2d.shape[0]
  nchunks = nstreams // (NW * KSTR)
  mesh = plsc.VectorSubcoreMesh(
      core_axis_name="c", subcore_axis_name="s", num_cores=NC,
      num_subcores=NSUB)
  return pl.kernel(
      functools.partial(_sc_gather_body, nchunks),
      out_type=jax.ShapeDtypeStruct((nstreams, BATCH, NS), jnp.float32),
      mesh=mesh,
      scratch_types=[
          pltpu.VMEM((KSTR, BATCH), jnp.int32),
          pltpu.VMEM((KSTR, BATCH, NS), jnp.float32),
          pltpu.SemaphoreType.DMA,
      ],
      compiler_params=pltpu.CompilerParams(use_tc_tiling_on_sc=False),
  )(node_attr, dst2d)


def _tc_dense_body(x_ref, ea_ref, sh_ref, w1_ref, b1_ref, w2_ref, b2_ref,
                   out_ref):
  ea = ea_ref[...]
  h = jnp.maximum(
      jnp.dot(ea, w1_ref[...], preferred_element_type=jnp.float32)
      + b1_ref[0:1, :], 0.0)
  wp = (jnp.dot(h, w2_ref[...], preferred_element_type=jnp.float32)
        + b2_ref[0:1, :])
  x = x_ref[...]
  acc = x[:, 0:1] * wp[:, 0:32]
  for i in range(1, NS):
    acc = acc + x[:, i:i + 1] * wp[:, 32 * i:32 * i + 32]
  sh = sh_ref[...]
  be = ea.shape[0]
  out_s = acc[:, :NS] * (sh[:, 0:1] * NORM)
  t = acc[:, NS:NS + NV] * NORM
  pieces = [out_s]
  for j in range(NV):
    pieces.append(t[:, j:j + 1] * sh[:, 1:4])
  pieces.append(jnp.ones((be, 1), jnp.float32))
  pieces.append(jnp.zeros((be, 3), jnp.float32))
  out_ref[...] = jnp.concatenate(pieces, axis=1)


def _tc_dense(x, edge_attr, edge_sh, w1, b1_8, w2p, b2p_8, block_e):
  e = x.shape[0]
  grid = (e // block_e,)
  return pl.pallas_call(
      _tc_dense_body,
      grid=grid,
      in_specs=[
          pl.BlockSpec((block_e, NS), lambda i: (i, 0)),
          pl.BlockSpec((block_e, NFEAT), lambda i: (i, 0)),
          pl.BlockSpec((block_e, 9), lambda i: (i, 0)),
          pl.BlockSpec((NFEAT, NFEAT), lambda i: (0, 0)),
          pl.BlockSpec((8, NFEAT), lambda i: (0, 0)),
          pl.BlockSpec((NFEAT, WPAD), lambda i: (0, 0)),
          pl.BlockSpec((8, WPAD), lambda i: (0, 0)),
      ],
      out_specs=pl.BlockSpec((block_e, TPW), lambda i: (i, 0)),
      out_shape=jax.ShapeDtypeStruct((e, TPW), jnp.float32),
      compiler_params=pltpu.CompilerParams(
          dimension_semantics=("arbitrary",)),
  )(x, edge_attr, edge_sh, w1, b1_8, w2p, b2p_8)


def _sc_scatter_body(nchunks, n_pad, tp_hbm, src_hbm, zero_hbm, out_hbm,
                     idx_v, tp_v, acc, sem):
  c = lax.axis_index("c")
  s = lax.axis_index("s")
  wid = s * NC + c
  rows_per_sub = n_pad // NSUB

  pltpu.sync_copy(zero_hbm.at[pl.ds(s * rows_per_sub, rows_per_sub)],
                  acc.at[pl.ds(s * rows_per_sub, rows_per_sub)])
  plsc.subcore_barrier()

  def chunk(j, carry):
    srow = (wid * nchunks + j) * KSTR
    pltpu.sync_copy(src_hbm.at[pl.ds(srow, KSTR)], idx_v)
    for half in range(2):
      pltpu.sync_copy(tp_hbm.at[pl.ds(srow + half * (KSTR // 2), KSTR // 2)],
                      tp_v)
      for jj in range(KSTR // 2):
        pltpu.sync_copy(tp_v.at[jj],
                        acc.at[idx_v.at[half * (KSTR // 2) + jj]], add=True)
    return carry

  lax.fori_loop(0, nchunks, chunk, 0)
  plsc.subcore_barrier()
  pltpu.sync_copy(acc.at[pl.ds(s * rows_per_sub, rows_per_sub)],
                  out_hbm.at[c, pl.ds(s * rows_per_sub, rows_per_sub)])


def _sc_scatter(tp3d, src2d, zeros_n):
  nstreams = tp3d.shape[0]
  nchunks = nstreams // (NW * KSTR)
  n_pad = zeros_n.shape[0]
  mesh = plsc.VectorSubcoreMesh(
      core_axis_name="c", subcore_axis_name="s", num_cores=NC,
      num_subcores=NSUB)
  return pl.kernel(
      functools.partial(_sc_scatter_body, nchunks, n_pad),
      out_type=jax.ShapeDtypeStruct((NC, n_pad, TPW), jnp.float32),
      mesh=mesh,
      scratch_types=[
          pltpu.VMEM((KSTR, BATCH), jnp.int32),
          pltpu.VMEM((KSTR // 2, BATCH, TPW), jnp.float32),
          pltpu.VMEM_SHARED((n_pad, TPW), jnp.float32),
          pltpu.SemaphoreType.DMA,
      ],
      compiler_params=pltpu.CompilerParams(use_tc_tiling_on_sc=False),
  )(tp3d, src2d, zeros_n)


def _tc_final_body(p0_ref, p1_ref, out_ref):
  tot = p0_ref[...] + p1_ref[...]
  cnt = jnp.maximum(tot[:, 28:29], 1.0)
  out_ref[...] = tot[:, :28] / cnt


def _tc_final(p0, p1, n, block_n):
  return pl.pallas_call(
      _tc_final_body,
      grid=(n // block_n,),
      in_specs=[
          pl.BlockSpec((block_n, TPW), lambda i: (i, 0)),
          pl.BlockSpec((block_n, TPW), lambda i: (i, 0)),
      ],
      out_specs=pl.BlockSpec((block_n, 28), lambda i: (i, 0)),
      out_shape=jax.ShapeDtypeStruct((n, 28), jnp.float32),
      compiler_params=pltpu.CompilerParams(
          dimension_semantics=("arbitrary",)),
  )(p0, p1)


def _rearrange_w2(W_fc2, b_fc2):
  src = np.zeros((WPAD,), dtype=np.int32)
  msk = np.zeros((WPAD,), dtype=np.float32)
  for i in range(NS):
    for j in range(NS):
      src[32 * i + j] = NS * i + j
      msk[32 * i + j] = 1.0
    for j in range(NV):
      src[32 * i + NS + j] = W0N + NV * i + j
      msk[32 * i + NS + j] = 1.0
  w2p = W_fc2[:, src] * msk[None, :]
  b2p = b_fc2[src] * msk
  return w2p, b2p


def kernel(node_attr, edge_index, edge_attr, edge_sh, W_fc1, b_fc1, W_fc2,
           b_fc2):
  e = edge_attr.shape[0]
  n = node_attr.shape[0]
  src = edge_index[0].astype(jnp.int32)
  dst = edge_index[1].astype(jnp.int32)
  dst2d = dst.reshape(e // BATCH, BATCH)
  src2d = src.reshape(e // BATCH, BATCH)

  w2p, b2p = _rearrange_w2(W_fc2, b_fc2)
  b1_8 = jnp.broadcast_to(b_fc1[None, :], (8, NFEAT))
  b2p_8 = jnp.broadcast_to(b2p[None, :], (8, WPAD))

  x3d = _sc_gather(node_attr, dst2d)
  x = x3d.reshape(e, NS)
  tp = _tc_dense(x, edge_attr, edge_sh, W_fc1, b1_8, w2p, b2p_8,
                 block_e=2000)
  n_pad = ((n // NSUB + 7) // 8 * 8) * NSUB
  zeros_n = jnp.zeros((n_pad, TPW), jnp.float32)
  partials = _sc_scatter(tp.reshape(e // BATCH, BATCH, TPW), src2d, zeros_n)
  out = _tc_final(partials[0], partials[1], n, block_n=2000)
  return out


# speedup vs baseline: 3.1358x; 2.4544x over previous
import functools

import jax
import jax.numpy as jnp
import numpy as np
from jax import lax
from jax.experimental import pallas as pl
from jax.experimental.pallas import tpu as pltpu
from jax.experimental.pallas import tpu_sc as plsc

NS = 16
NV = 4
NFEAT = 48
W0N = NS * NS
W1N = NS * NV
WPAD = 512
NORM = 1.0 / np.sqrt(float(NS))

NC = 2
NSUB = 16
NW = NC * NSUB

BATCH = 125
KSTR = 8
CHUNK = BATCH * KSTR

TPW = 32


def _sc_gather_body(nchunks, node_hbm, idx_hbm, out_hbm, idx_v, rows_v, sem):
  c = lax.axis_index("c")
  s = lax.axis_index("s")
  wid = s * NC + c

  def chunk(j, carry):
    srow = (wid * nchunks + j) * KSTR
    pltpu.sync_copy(idx_hbm.at[pl.ds(srow, KSTR)], idx_v)
    cps = [
        pltpu.async_copy(node_hbm.at[idx_v.at[jj]], rows_v.at[jj], sem)
        for jj in range(KSTR)
    ]
    for cp in cps:
      cp.wait()
    pltpu.sync_copy(rows_v, out_hbm.at[pl.ds(srow, KSTR)])
    return carry

  lax.fori_loop(0, nchunks, chunk, 0)


def _sc_gather(node_attr, dst2d):
  nstreams = dst2d.shape[0]
  nchunks = nstreams // (NW * KSTR)
  mesh = plsc.VectorSubcoreMesh(
      core_axis_name="c", subcore_axis_name="s", num_cores=NC,
      num_subcores=NSUB)
  return pl.kernel(
      functools.partial(_sc_gather_body, nchunks),
      out_type=jax.ShapeDtypeStruct((nstreams, BATCH, NS), jnp.float32),
      mesh=mesh,
      scratch_types=[
          pltpu.VMEM((KSTR, BATCH), jnp.int32),
          pltpu.VMEM((KSTR, BATCH, NS), jnp.float32),
          pltpu.SemaphoreType.DMA,
      ],
      compiler_params=pltpu.CompilerParams(use_tc_tiling_on_sc=False),
  )(node_attr, dst2d)


def _tc_dense_body(x_ref, ea_ref, sh_ref, w1_ref, b1_ref, w2_ref, b2_ref,
                   r_ref, f_ref, m_ref, out_ref):
  ea = ea_ref[...]
  h = jnp.maximum(
      jnp.dot(ea, w1_ref[...], preferred_element_type=jnp.float32)
      + b1_ref[0:1, :], 0.0)
  wp = (jnp.dot(h, w2_ref[...], preferred_element_type=jnp.float32)
        + b2_ref[0:1, :])
  xbig = jnp.dot(x_ref[...], r_ref[...], preferred_element_type=jnp.float32)
  pre = jnp.dot(wp * xbig, f_ref[...], preferred_element_type=jnp.float32)
  be = ea.shape[0]
  sh16 = jnp.concatenate(
      [sh_ref[...], jnp.zeros((be, 7), jnp.float32)], axis=1)
  shf = jnp.dot(sh16, m_ref[...], preferred_element_type=jnp.float32)
  out_ref[...] = jnp.concatenate(
      [pre * shf, jnp.ones((be, 1), jnp.float32),
       jnp.zeros((be, 3), jnp.float32)], axis=1)


def _tc_dense(x, edge_attr, edge_sh, w1, b1_8, w2p, b2p_8, rmat, fmat, mmat,
              block_e):
  e = x.shape[0]
  grid = (e // block_e,)
  return pl.pallas_call(
      _tc_dense_body,
      grid=grid,
      in_specs=[
          pl.BlockSpec((block_e, NS), lambda i: (i, 0)),
          pl.BlockSpec((block_e, NFEAT), lambda i: (i, 0)),
          pl.BlockSpec((block_e, 9), lambda i: (i, 0)),
          pl.BlockSpec((NFEAT, NFEAT), lambda i: (0, 0)),
          pl.BlockSpec((8, NFEAT), lambda i: (0, 0)),
          pl.BlockSpec((NFEAT, WPAD), lambda i: (0, 0)),
          pl.BlockSpec((8, WPAD), lambda i: (0, 0)),
          pl.BlockSpec((NS, WPAD), lambda i: (0, 0)),
          pl.BlockSpec((WPAD, 28), lambda i: (0, 0)),
          pl.BlockSpec((NS, 28), lambda i: (0, 0)),
      ],
      out_specs=pl.BlockSpec((block_e, TPW), lambda i: (i, 0)),
      out_shape=jax.ShapeDtypeStruct((e, TPW), jnp.float32),
      compiler_params=pltpu.CompilerParams(
          dimension_semantics=("arbitrary",)),
  )(x, edge_attr, edge_sh, w1, b1_8, w2p, b2p_8, rmat, fmat, mmat)


def _sc_scatter_body(nchunks, n_pad, tp_hbm, src_hbm, zero_hbm, out_hbm,
                     idx_v, tp_v, acc, sem):
  c = lax.axis_index("c")
  s = lax.axis_index("s")
  wid = s * NC + c
  rows_per_sub = n_pad // NSUB

  pltpu.sync_copy(zero_hbm.at[pl.ds(s * rows_per_sub, rows_per_sub)],
                  acc.at[pl.ds(s * rows_per_sub, rows_per_sub)])
  plsc.subcore_barrier()

  def chunk(j, carry):
    srow = (wid * nchunks + j) * KSTR
    pltpu.sync_copy(src_hbm.at[pl.ds(srow, KSTR)], idx_v)
    for half in range(2):
      pltpu.sync_copy(tp_hbm.at[pl.ds(srow + half * (KSTR // 2), KSTR // 2)],
                      tp_v)
      for jj in range(KSTR // 2):
        pltpu.sync_copy(tp_v.at[jj],
                        acc.at[idx_v.at[half * (KSTR // 2) + jj]], add=True)
    return carry

  lax.fori_loop(0, nchunks, chunk, 0)
  plsc.subcore_barrier()
  pltpu.sync_copy(acc.at[pl.ds(s * rows_per_sub, rows_per_sub)],
                  out_hbm.at[c, pl.ds(s * rows_per_sub, rows_per_sub)])


def _sc_scatter(tp3d, src2d, zeros_n):
  nstreams = tp3d.shape[0]
  nchunks = nstreams // (NW * KSTR)
  n_pad = zeros_n.shape[0]
  mesh = plsc.VectorSubcoreMesh(
      core_axis_name="c", subcore_axis_name="s", num_cores=NC,
      num_subcores=NSUB)
  return pl.kernel(
      functools.partial(_sc_scatter_body, nchunks, n_pad),
      out_type=jax.ShapeDtypeStruct((NC, n_pad, TPW), jnp.float32),
      mesh=mesh,
      scratch_types=[
          pltpu.VMEM((KSTR, BATCH), jnp.int32),
          pltpu.VMEM((KSTR // 2, BATCH, TPW), jnp.float32),
          pltpu.VMEM_SHARED((n_pad, TPW), jnp.float32),
          pltpu.SemaphoreType.DMA,
      ],
      compiler_params=pltpu.CompilerParams(use_tc_tiling_on_sc=False),
  )(tp3d, src2d, zeros_n)


def _tc_final_body(p0_ref, p1_ref, out_ref):
  tot = p0_ref[...] + p1_ref[...]
  cnt = jnp.maximum(tot[:, 28:29], 1.0)
  out_ref[...] = tot[:, :28] / cnt


def _tc_final(p0, p1, n, block_n):
  return pl.pallas_call(
      _tc_final_body,
      grid=(n // block_n,),
      in_specs=[
          pl.BlockSpec((block_n, TPW), lambda i: (i, 0)),
          pl.BlockSpec((block_n, TPW), lambda i: (i, 0)),
      ],
      out_specs=pl.BlockSpec((block_n, 28), lambda i: (i, 0)),
      out_shape=jax.ShapeDtypeStruct((n, 28), jnp.float32),
      compiler_params=pltpu.CompilerParams(
          dimension_semantics=("arbitrary",)),
  )(p0, p1)


def _rearrange_w2(W_fc2, b_fc2):
  src = np.zeros((WPAD,), dtype=np.int32)
  msk = np.zeros((WPAD,), dtype=np.float32)
  for i in range(NS):
    for j in range(NS):
      src[32 * i + j] = NS * i + j
      msk[32 * i + j] = 1.0
    for j in range(NV):
      src[32 * i + NS + j] = W0N + NV * i + j
      msk[32 * i + NS + j] = 1.0
  w2p = W_fc2[:, src] * msk[None, :]
  b2p = b_fc2[src] * msk
  return w2p, b2p


def _const_mats():
  rmat = np.zeros((NS, WPAD), dtype=np.float32)
  fmat = np.zeros((WPAD, 28), dtype=np.float32)
  mmat = np.zeros((NS, 28), dtype=np.float32)
  for i in range(NS):
    for j in range(NS + NV):
      rmat[i, 32 * i + j] = 1.0
    for j in range(NS):
      fmat[32 * i + j, j] = NORM
    for j in range(NV):
      for c in range(3):
        fmat[32 * i + NS + j, NS + 3 * j + c] = NORM
  mmat[0, 0:NS] = 1.0
  for j in range(NV):
    for c in range(3):
      mmat[1 + c, NS + 3 * j + c] = 1.0
  return jnp.asarray(rmat), jnp.asarray(fmat), jnp.asarray(mmat)


def kernel(node_attr, edge_index, edge_attr, edge_sh, W_fc1, b_fc1, W_fc2,
           b_fc2):
  e = edge_attr.shape[0]
  n = node_attr.shape[0]
  src = edge_index[0].astype(jnp.int32)
  dst = edge_index[1].astype(jnp.int32)
  dst2d = dst.reshape(e // BATCH, BATCH)
  src2d = src.reshape(e // BATCH, BATCH)

  w2p, b2p = _rearrange_w2(W_fc2, b_fc2)
  b1_8 = jnp.broadcast_to(b_fc1[None, :], (8, NFEAT))
  b2p_8 = jnp.broadcast_to(b2p[None, :], (8, WPAD))

  rmat, fmat, mmat = _const_mats()
  x3d = _sc_gather(node_attr, dst2d)
  x = x3d.reshape(e, NS)
  tp = _tc_dense(x, edge_attr, edge_sh, W_fc1, b1_8, w2p, b2p_8,
                 rmat, fmat, mmat, block_e=2000)
  n_pad = ((n // NSUB + 7) // 8 * 8) * NSUB
  zeros_n = jnp.zeros((n_pad, TPW), jnp.float32)
  partials = _sc_scatter(tp.reshape(e // BATCH, BATCH, TPW), src2d, zeros_n)
  out = _tc_final(partials[0], partials[1], n, block_n=2000)
  return out


# speedup vs baseline: 3.2643x; 1.0410x over previous
import functools

import jax
import jax.numpy as jnp
import numpy as np
from jax import lax
from jax.experimental import pallas as pl
from jax.experimental.pallas import tpu as pltpu
from jax.experimental.pallas import tpu_sc as plsc

NS = 16
NV = 4
NFEAT = 48
W0N = NS * NS
W1N = NS * NV
WPAD = 512
NORM = 1.0 / np.sqrt(float(NS))

NC = 2
NSUB = 16
NW = NC * NSUB

BATCH = 125
KSTR = 8
CHUNK = BATCH * KSTR

TPW = 32


def _sc_gather_body(nchunks, node_hbm, idx_hbm, out_hbm, idx_v, rows_v, sem):
  c = lax.axis_index("c")
  s = lax.axis_index("s")
  wid = s * NC + c

  def chunk(j, carry):
    srow = (wid * nchunks + j) * KSTR
    pltpu.sync_copy(idx_hbm.at[pl.ds(srow, KSTR)], idx_v)
    cps = [
        pltpu.async_copy(node_hbm.at[idx_v.at[jj]], rows_v.at[jj], sem)
        for jj in range(KSTR)
    ]
    for cp in cps:
      cp.wait()
    pltpu.sync_copy(rows_v, out_hbm.at[pl.ds(srow, KSTR)])
    return carry

  lax.fori_loop(0, nchunks, chunk, 0)


def _sc_gather(node_attr, dst2d):
  nstreams = dst2d.shape[0]
  nchunks = nstreams // (NW * KSTR)
  mesh = plsc.VectorSubcoreMesh(
      core_axis_name="c", subcore_axis_name="s", num_cores=NC,
      num_subcores=NSUB)
  return pl.kernel(
      functools.partial(_sc_gather_body, nchunks),
      out_type=jax.ShapeDtypeStruct((nstreams, BATCH, NS), jnp.float32),
      mesh=mesh,
      scratch_types=[
          pltpu.VMEM((KSTR, BATCH), jnp.int32),
          pltpu.VMEM((KSTR, BATCH, NS), jnp.float32),
          pltpu.SemaphoreType.DMA,
      ],
      compiler_params=pltpu.CompilerParams(use_tc_tiling_on_sc=False),
  )(node_attr, dst2d)


def _tc_dense_body(x_ref, ea_ref, sh_ref, w1_ref, b1_ref, w2_ref, b2_ref,
                   r_ref, f_ref, m_ref, out_ref):
  ea = ea_ref[...]
  h = jnp.maximum(
      jnp.dot(ea, w1_ref[...], preferred_element_type=jnp.float32)
      + b1_ref[0:1, :], 0.0)
  wp = (jnp.dot(h, w2_ref[...], preferred_element_type=jnp.float32)
        + b2_ref[0:1, :])
  xbig = jnp.dot(x_ref[...], r_ref[...], preferred_element_type=jnp.float32)
  pre = jnp.dot(wp * xbig, f_ref[...], preferred_element_type=jnp.float32)
  be = ea.shape[0]
  sh16 = jnp.concatenate(
      [sh_ref[...], jnp.zeros((be, 7), jnp.float32)], axis=1)
  shf = jnp.dot(sh16, m_ref[...], preferred_element_type=jnp.float32)
  out_ref[...] = jnp.concatenate(
      [pre * shf, jnp.ones((be, 1), jnp.float32),
       jnp.zeros((be, 3), jnp.float32)], axis=1)


def _tc_dense(x, edge_attr, edge_sh, w1, b1_8, w2p, b2p_8, rmat, fmat, mmat,
              block_e):
  e = x.shape[0]
  grid = (e // block_e,)
  return pl.pallas_call(
      _tc_dense_body,
      grid=grid,
      in_specs=[
          pl.BlockSpec((block_e, NS), lambda i: (i, 0)),
          pl.BlockSpec((block_e, NFEAT), lambda i: (i, 0)),
          pl.BlockSpec((block_e, 9), lambda i: (i, 0)),
          pl.BlockSpec((NFEAT, NFEAT), lambda i: (0, 0)),
          pl.BlockSpec((8, NFEAT), lambda i: (0, 0)),
          pl.BlockSpec((NFEAT, WPAD), lambda i: (0, 0)),
          pl.BlockSpec((8, WPAD), lambda i: (0, 0)),
          pl.BlockSpec((NS, WPAD), lambda i: (0, 0)),
          pl.BlockSpec((WPAD, 28), lambda i: (0, 0)),
          pl.BlockSpec((NS, 28), lambda i: (0, 0)),
      ],
      out_specs=pl.BlockSpec((block_e, TPW), lambda i: (i, 0)),
      out_shape=jax.ShapeDtypeStruct((e, TPW), jnp.float32),
      compiler_params=pltpu.CompilerParams(
          dimension_semantics=("arbitrary",)),
  )(x, edge_attr, edge_sh, w1, b1_8, w2p, b2p_8, rmat, fmat, mmat)


def _sc_scatter_body(nchunks, n_pad, tp_hbm, src_hbm, zero_hbm, out_hbm,
                     idx_v, tp_v, acc, sem):
  c = lax.axis_index("c")
  s = lax.axis_index("s")
  wid = s * NC + c
  rows_per_sub = n_pad // NSUB

  pltpu.sync_copy(zero_hbm.at[pl.ds(s * rows_per_sub, rows_per_sub)],
                  acc.at[pl.ds(s * rows_per_sub, rows_per_sub)])
  plsc.subcore_barrier()

  def chunk(j, carry):
    srow = (wid * nchunks + j) * KSTR
    pltpu.sync_copy(src_hbm.at[pl.ds(srow, KSTR)], idx_v)
    for half in range(2):
      pltpu.sync_copy(tp_hbm.at[pl.ds(srow + half * (KSTR // 2), KSTR // 2)],
                      tp_v)
      cps = [
          pltpu.async_copy(tp_v.at[jj],
                           acc.at[idx_v.at[half * (KSTR // 2) + jj]], sem,
                           add=True)
          for jj in range(KSTR // 2)
      ]
      for cp in cps:
        cp.wait()
    return carry

  lax.fori_loop(0, nchunks, chunk, 0)
  plsc.subcore_barrier()
  pltpu.sync_copy(acc.at[pl.ds(s * rows_per_sub, rows_per_sub)],
                  out_hbm.at[c, pl.ds(s * rows_per_sub, rows_per_sub)])


def _sc_scatter(tp3d, src2d, zeros_n):
  nstreams = tp3d.shape[0]
  nchunks = nstreams // (NW * KSTR)
  n_pad = zeros_n.shape[0]
  mesh = plsc.VectorSubcoreMesh(
      core_axis_name="c", subcore_axis_name="s", num_cores=NC,
      num_subcores=NSUB)
  return pl.kernel(
      functools.partial(_sc_scatter_body, nchunks, n_pad),
      out_type=jax.ShapeDtypeStruct((NC, n_pad, TPW), jnp.float32),
      mesh=mesh,
      scratch_types=[
          pltpu.VMEM((KSTR, BATCH), jnp.int32),
          pltpu.VMEM((KSTR // 2, BATCH, TPW), jnp.float32),
          pltpu.VMEM_SHARED((n_pad, TPW), jnp.float32),
          pltpu.SemaphoreType.DMA,
      ],
      compiler_params=pltpu.CompilerParams(use_tc_tiling_on_sc=False),
  )(tp3d, src2d, zeros_n)


def _tc_final_body(p0_ref, p1_ref, out_ref):
  tot = p0_ref[...] + p1_ref[...]
  cnt = jnp.maximum(tot[:, 28:29], 1.0)
  out_ref[...] = tot[:, :28] / cnt


def _tc_final(p0, p1, n, block_n):
  return pl.pallas_call(
      _tc_final_body,
      grid=(n // block_n,),
      in_specs=[
          pl.BlockSpec((block_n, TPW), lambda i: (i, 0)),
          pl.BlockSpec((block_n, TPW), lambda i: (i, 0)),
      ],
      out_specs=pl.BlockSpec((block_n, 28), lambda i: (i, 0)),
      out_shape=jax.ShapeDtypeStruct((n, 28), jnp.float32),
      compiler_params=pltpu.CompilerParams(
          dimension_semantics=("arbitrary",)),
  )(p0, p1)


def _rearrange_w2(W_fc2, b_fc2):
  src = np.zeros((WPAD,), dtype=np.int32)
  msk = np.zeros((WPAD,), dtype=np.float32)
  for i in range(NS):
    for j in range(NS):
      src[32 * i + j] = NS * i + j
      msk[32 * i + j] = 1.0
    for j in range(NV):
      src[32 * i + NS + j] = W0N + NV * i + j
      msk[32 * i + NS + j] = 1.0
  w2p = W_fc2[:, src] * msk[None, :]
  b2p = b_fc2[src] * msk
  return w2p, b2p


def _const_mats():
  rmat = np.zeros((NS, WPAD), dtype=np.float32)
  fmat = np.zeros((WPAD, 28), dtype=np.float32)
  mmat = np.zeros((NS, 28), dtype=np.float32)
  for i in range(NS):
    for j in range(NS + NV):
      rmat[i, 32 * i + j] = 1.0
    for j in range(NS):
      fmat[32 * i + j, j] = NORM
    for j in range(NV):
      for c in range(3):
        fmat[32 * i + NS + j, NS + 3 * j + c] = NORM
  mmat[0, 0:NS] = 1.0
  for j in range(NV):
    for c in range(3):
      mmat[1 + c, NS + 3 * j + c] = 1.0
  return jnp.asarray(rmat), jnp.asarray(fmat), jnp.asarray(mmat)


def kernel(node_attr, edge_index, edge_attr, edge_sh, W_fc1, b_fc1, W_fc2,
           b_fc2):
  e = edge_attr.shape[0]
  n = node_attr.shape[0]
  src = edge_index[0].astype(jnp.int32)
  dst = edge_index[1].astype(jnp.int32)
  dst2d = dst.reshape(e // BATCH, BATCH)
  src2d = src.reshape(e // BATCH, BATCH)

  w2p, b2p = _rearrange_w2(W_fc2, b_fc2)
  b1_8 = jnp.broadcast_to(b_fc1[None, :], (8, NFEAT))
  b2p_8 = jnp.broadcast_to(b2p[None, :], (8, WPAD))

  rmat, fmat, mmat = _const_mats()
  x3d = _sc_gather(node_attr, dst2d)
  x = x3d.reshape(e, NS)
  tp = _tc_dense(x, edge_attr, edge_sh, W_fc1, b1_8, w2p, b2p_8,
                 rmat, fmat, mmat, block_e=4000)
  n_pad = ((n // NSUB + 7) // 8 * 8) * NSUB
  zeros_n = jnp.zeros((n_pad, TPW), jnp.float32)
  partials = _sc_scatter(tp.reshape(e // BATCH, BATCH, TPW), src2d, zeros_n)
  out = _tc_final(partials[0], partials[1], n, block_n=2000)
  return out


# speedup vs baseline: 3.3017x; 1.0115x over previous
import functools

import jax
import jax.numpy as jnp
import numpy as np
from jax import lax
from jax.experimental import pallas as pl
from jax.experimental.pallas import tpu as pltpu
from jax.experimental.pallas import tpu_sc as plsc

NS = 16
NV = 4
NFEAT = 48
W0N = NS * NS
W1N = NS * NV
WPAD = 512
NORM = 1.0 / np.sqrt(float(NS))

NC = 2
NSUB = 16
NW = NC * NSUB

BATCH = 125
KSTR = 8
CHUNK = BATCH * KSTR

TPW = 32


def _sc_gather_body(nchunks, node_hbm, idx_hbm, out_hbm, idx_v, rows_v, sem):
  c = lax.axis_index("c")
  s = lax.axis_index("s")
  wid = s * NC + c

  def chunk(j, carry):
    srow = (wid * nchunks + j) * KSTR
    pltpu.sync_copy(idx_hbm.at[pl.ds(srow, KSTR)], idx_v)
    cps = [
        pltpu.async_copy(node_hbm.at[idx_v.at[jj]], rows_v.at[jj], sem)
        for jj in range(KSTR)
    ]
    for cp in cps:
      cp.wait()
    pltpu.sync_copy(rows_v, out_hbm.at[pl.ds(srow, KSTR)])
    return carry

  lax.fori_loop(0, nchunks, chunk, 0)


def _sc_gather(node_attr, dst2d):
  nstreams = dst2d.shape[0]
  nchunks = nstreams // (NW * KSTR)
  mesh = plsc.VectorSubcoreMesh(
      core_axis_name="c", subcore_axis_name="s", num_cores=NC,
      num_subcores=NSUB)
  return pl.kernel(
      functools.partial(_sc_gather_body, nchunks),
      out_type=jax.ShapeDtypeStruct((nstreams, BATCH, NS), jnp.float32),
      mesh=mesh,
      scratch_types=[
          pltpu.VMEM((KSTR, BATCH), jnp.int32),
          pltpu.VMEM((KSTR, BATCH, NS), jnp.float32),
          pltpu.SemaphoreType.DMA,
      ],
      compiler_params=pltpu.CompilerParams(use_tc_tiling_on_sc=False),
  )(node_attr, dst2d)


def _tc_dense_body(x_ref, ea_ref, sh_ref, w1_ref, b1_ref, w2_ref, b2_ref,
                   r_ref, f_ref, m_ref, out_ref):
  ea = ea_ref[...]
  h = jnp.maximum(
      jnp.dot(ea, w1_ref[...], preferred_element_type=jnp.float32)
      + b1_ref[0:1, :], 0.0)
  wp = (jnp.dot(h, w2_ref[...], preferred_element_type=jnp.float32)
        + b2_ref[0:1, :])
  xbig = jnp.dot(x_ref[...], r_ref[...], preferred_element_type=jnp.float32)
  pre = jnp.dot(wp * xbig, f_ref[...], preferred_element_type=jnp.float32)
  be = ea.shape[0]
  sh16 = jnp.concatenate(
      [sh_ref[...], jnp.zeros((be, 7), jnp.float32)], axis=1)
  shf = jnp.dot(sh16, m_ref[...], preferred_element_type=jnp.float32)
  out_ref[...] = jnp.concatenate(
      [pre * shf, jnp.ones((be, 1), jnp.float32),
       jnp.zeros((be, 3), jnp.float32)], axis=1)


def _tc_dense(x, edge_attr, edge_sh, w1, b1_8, w2p, b2p_8, rmat, fmat, mmat,
              block_e):
  e = x.shape[0]
  grid = (e // block_e,)
  return pl.pallas_call(
      _tc_dense_body,
      grid=grid,
      in_specs=[
          pl.BlockSpec((block_e, NS), lambda i: (i, 0)),
          pl.BlockSpec((block_e, NFEAT), lambda i: (i, 0)),
          pl.BlockSpec((block_e, 9), lambda i: (i, 0)),
          pl.BlockSpec((NFEAT, NFEAT), lambda i: (0, 0)),
          pl.BlockSpec((8, NFEAT), lambda i: (0, 0)),
          pl.BlockSpec((NFEAT, WPAD), lambda i: (0, 0)),
          pl.BlockSpec((8, WPAD), lambda i: (0, 0)),
          pl.BlockSpec((NS, WPAD), lambda i: (0, 0)),
          pl.BlockSpec((WPAD, 28), lambda i: (0, 0)),
          pl.BlockSpec((NS, 28), lambda i: (0, 0)),
      ],
      out_specs=pl.BlockSpec((block_e, TPW), lambda i: (i, 0)),
      out_shape=jax.ShapeDtypeStruct((e, TPW), jnp.float32),
      compiler_params=pltpu.CompilerParams(
          dimension_semantics=("arbitrary",)),
  )(x, edge_attr, edge_sh, w1, b1_8, w2p, b2p_8, rmat, fmat, mmat)


def _sc_scatter_body(nchunks, n_pad, tp_hbm, src_hbm, zero_hbm, out_hbm,
                     idx_v, tp_v, acc, sem):
  c = lax.axis_index("c")
  s = lax.axis_index("s")
  wid = s * NC + c
  rows_per_sub = n_pad // NSUB

  pltpu.sync_copy(zero_hbm.at[pl.ds(s * rows_per_sub, rows_per_sub)],
                  acc.at[pl.ds(s * rows_per_sub, rows_per_sub)])
  plsc.subcore_barrier()

  def chunk(j, carry):
    srow = (wid * nchunks + j) * KSTR
    pltpu.sync_copy(src_hbm.at[pl.ds(srow, KSTR)], idx_v)
    for half in range(2):
      pltpu.sync_copy(tp_hbm.at[pl.ds(srow + half * (KSTR // 2), KSTR // 2)],
                      tp_v)
      cps = [
          pltpu.async_copy(tp_v.at[jj],
                           acc.at[idx_v.at[half * (KSTR // 2) + jj]], sem,
                           add=True)
          for jj in range(KSTR // 2)
      ]
      for cp in cps:
        cp.wait()
    return carry

  lax.fori_loop(0, nchunks, chunk, 0)
  plsc.subcore_barrier()
  pltpu.sync_copy(acc.at[pl.ds(s * rows_per_sub, rows_per_sub)],
                  out_hbm.at[c, pl.ds(s * rows_per_sub, rows_per_sub)])


def _sc_scatter(tp3d, src2d, zeros_n):
  nstreams = tp3d.shape[0]
  nchunks = nstreams // (NW * KSTR)
  n_pad = zeros_n.shape[0]
  mesh = plsc.VectorSubcoreMesh(
      core_axis_name="c", subcore_axis_name="s", num_cores=NC,
      num_subcores=NSUB)
  return pl.kernel(
      functools.partial(_sc_scatter_body, nchunks, n_pad),
      out_type=jax.ShapeDtypeStruct((NC, n_pad, TPW), jnp.float32),
      mesh=mesh,
      scratch_types=[
          pltpu.VMEM((KSTR, BATCH), jnp.int32),
          pltpu.VMEM((KSTR // 2, BATCH, TPW), jnp.float32),
          pltpu.VMEM_SHARED((n_pad, TPW), jnp.float32),
          pltpu.SemaphoreType.DMA,
      ],
      compiler_params=pltpu.CompilerParams(use_tc_tiling_on_sc=False),
  )(tp3d, src2d, zeros_n)


def _tc_final_body(p0_ref, p1_ref, out_ref):
  tot = p0_ref[...] + p1_ref[...]
  cnt = jnp.maximum(tot[:, 28:29], 1.0)
  out_ref[...] = tot[:, :28] / cnt


def _tc_final(p0, p1, n, block_n):
  return pl.pallas_call(
      _tc_final_body,
      grid=(n // block_n,),
      in_specs=[
          pl.BlockSpec((block_n, TPW), lambda i: (i, 0)),
          pl.BlockSpec((block_n, TPW), lambda i: (i, 0)),
      ],
      out_specs=pl.BlockSpec((block_n, 28), lambda i: (i, 0)),
      out_shape=jax.ShapeDtypeStruct((n, 28), jnp.float32),
      compiler_params=pltpu.CompilerParams(
          dimension_semantics=("arbitrary",)),
  )(p0, p1)


def _rearrange_w2(W_fc2, b_fc2):
  src = np.zeros((WPAD,), dtype=np.int32)
  msk = np.zeros((WPAD,), dtype=np.float32)
  for i in range(NS):
    for j in range(NS):
      src[32 * i + j] = NS * i + j
      msk[32 * i + j] = 1.0
    for j in range(NV):
      src[32 * i + NS + j] = W0N + NV * i + j
      msk[32 * i + NS + j] = 1.0
  w2p = W_fc2[:, src] * msk[None, :]
  b2p = b_fc2[src] * msk
  return w2p, b2p


def _const_mats():
  rmat = np.zeros((NS, WPAD), dtype=np.float32)
  fmat = np.zeros((WPAD, 28), dtype=np.float32)
  mmat = np.zeros((NS, 28), dtype=np.float32)
  for i in range(NS):
    for j in range(NS + NV):
      rmat[i, 32 * i + j] = 1.0
    for j in range(NS):
      fmat[32 * i + j, j] = NORM
    for j in range(NV):
      for c in range(3):
        fmat[32 * i + NS + j, NS + 3 * j + c] = NORM
  mmat[0, 0:NS] = 1.0
  for j in range(NV):
    for c in range(3):
      mmat[1 + c, NS + 3 * j + c] = 1.0
  return jnp.asarray(rmat), jnp.asarray(fmat), jnp.asarray(mmat)


def kernel(node_attr, edge_index, edge_attr, edge_sh, W_fc1, b_fc1, W_fc2,
           b_fc2):
  e = edge_attr.shape[0]
  n = node_attr.shape[0]
  src = edge_index[0].astype(jnp.int32)
  dst = edge_index[1].astype(jnp.int32)
  dst2d = dst.reshape(e // BATCH, BATCH)
  src2d = src.reshape(e // BATCH, BATCH)

  w2p, b2p = _rearrange_w2(W_fc2, b_fc2)
  b1_8 = jnp.broadcast_to(b_fc1[None, :], (8, NFEAT))
  b2p_8 = jnp.broadcast_to(b2p[None, :], (8, WPAD))

  rmat, fmat, mmat = _const_mats()
  x3d = _sc_gather(node_attr, dst2d)
  x = x3d.reshape(e, NS)
  tp = _tc_dense(x, edge_attr, edge_sh, W_fc1, b1_8, w2p, b2p_8,
                 rmat, fmat, mmat, block_e=8000)
  n_pad = ((n // NSUB + 7) // 8 * 8) * NSUB
  zeros_n = jnp.zeros((n_pad, TPW), jnp.float32)
  partials = _sc_scatter(tp.reshape(e // BATCH, BATCH, TPW), src2d, zeros_n)
  out = _tc_final(partials[0], partials[1], n, block_n=2000)
  return out


# speedup vs baseline: 3.6598x; 1.1085x over previous
import functools

import jax
import jax.numpy as jnp
import numpy as np
from jax import lax
from jax.experimental import pallas as pl
from jax.experimental.pallas import tpu as pltpu
from jax.experimental.pallas import tpu_sc as plsc

NS = 16
NV = 4
NFEAT = 48
W0N = NS * NS
W1N = NS * NV
WPAD = 320
NORM = 1.0 / np.sqrt(float(NS))

NC = 2
NSUB = 16
NW = NC * NSUB

BATCH = 125
KSTR = 8
CHUNK = BATCH * KSTR

TPW = 32


def _sc_gather_body(nchunks, node_hbm, idx_hbm, out_hbm, idx_v, rows_v, sem):
  c = lax.axis_index("c")
  s = lax.axis_index("s")
  wid = s * NC + c

  def chunk(j, carry):
    srow = (wid * nchunks + j) * KSTR
    pltpu.sync_copy(idx_hbm.at[pl.ds(srow, KSTR)], idx_v)
    cps = [
        pltpu.async_copy(node_hbm.at[idx_v.at[jj]], rows_v.at[jj], sem)
        for jj in range(KSTR)
    ]
    for cp in cps:
      cp.wait()
    pltpu.sync_copy(rows_v, out_hbm.at[pl.ds(srow, KSTR)])
    return carry

  lax.fori_loop(0, nchunks, chunk, 0)


def _sc_gather(node_attr, dst2d):
  nstreams = dst2d.shape[0]
  nchunks = nstreams // (NW * KSTR)
  mesh = plsc.VectorSubcoreMesh(
      core_axis_name="c", subcore_axis_name="s", num_cores=NC,
      num_subcores=NSUB)
  return pl.kernel(
      functools.partial(_sc_gather_body, nchunks),
      out_type=jax.ShapeDtypeStruct((nstreams, BATCH, NS), jnp.float32),
      mesh=mesh,
      scratch_types=[
          pltpu.VMEM((KSTR, BATCH), jnp.int32),
          pltpu.VMEM((KSTR, BATCH, NS), jnp.float32),
          pltpu.SemaphoreType.DMA,
      ],
      compiler_params=pltpu.CompilerParams(use_tc_tiling_on_sc=False),
  )(node_attr, dst2d)


def _tc_dense_body(x_ref, ea_ref, sh_ref, w1_ref, b1_ref, w2_ref, b2_ref,
                   r_ref, f_ref, m_ref, out_ref):
  ea = ea_ref[...]
  h = jnp.maximum(
      jnp.dot(ea, w1_ref[...], preferred_element_type=jnp.float32)
      + b1_ref[0:1, :], 0.0)
  wp = (jnp.dot(h, w2_ref[...], preferred_element_type=jnp.float32)
        + b2_ref[0:1, :])
  xbig = jnp.dot(x_ref[...], r_ref[...], preferred_element_type=jnp.float32)
  pre = jnp.dot(wp * xbig, f_ref[...], preferred_element_type=jnp.float32)
  be = ea.shape[0]
  sh16 = jnp.concatenate(
      [sh_ref[...], jnp.zeros((be, 7), jnp.float32)], axis=1)
  shf = jnp.dot(sh16, m_ref[...], preferred_element_type=jnp.float32)
  out_ref[...] = jnp.concatenate(
      [pre * shf, jnp.ones((be, 1), jnp.float32),
       jnp.zeros((be, 3), jnp.float32)], axis=1)


def _tc_dense(x, edge_attr, edge_sh, w1, b1_8, w2p, b2p_8, rmat, fmat, mmat,
              block_e):
  e = x.shape[0]
  grid = (e // block_e,)
  return pl.pallas_call(
      _tc_dense_body,
      grid=grid,
      in_specs=[
          pl.BlockSpec((block_e, NS), lambda i: (i, 0)),
          pl.BlockSpec((block_e, NFEAT), lambda i: (i, 0)),
          pl.BlockSpec((block_e, 9), lambda i: (i, 0)),
          pl.BlockSpec((NFEAT, NFEAT), lambda i: (0, 0)),
          pl.BlockSpec((8, NFEAT), lambda i: (0, 0)),
          pl.BlockSpec((NFEAT, WPAD), lambda i: (0, 0)),
          pl.BlockSpec((8, WPAD), lambda i: (0, 0)),
          pl.BlockSpec((NS, WPAD), lambda i: (0, 0)),
          pl.BlockSpec((WPAD, 28), lambda i: (0, 0)),
          pl.BlockSpec((NS, 28), lambda i: (0, 0)),
      ],
      out_specs=pl.BlockSpec((block_e, TPW), lambda i: (i, 0)),
      out_shape=jax.ShapeDtypeStruct((e, TPW), jnp.float32),
      compiler_params=pltpu.CompilerParams(
          dimension_semantics=("arbitrary",)),
  )(x, edge_attr, edge_sh, w1, b1_8, w2p, b2p_8, rmat, fmat, mmat)


def _sc_scatter_body(nchunks, n_pad, tp_hbm, src_hbm, zero_hbm, out_hbm,
                     idx_v, tp_v, acc, sem):
  c = lax.axis_index("c")
  s = lax.axis_index("s")
  wid = s * NC + c
  rows_per_sub = n_pad // NSUB

  pltpu.sync_copy(zero_hbm.at[pl.ds(s * rows_per_sub, rows_per_sub)],
                  acc.at[pl.ds(s * rows_per_sub, rows_per_sub)])
  plsc.subcore_barrier()

  def chunk(j, carry):
    srow = (wid * nchunks + j) * KSTR
    pltpu.sync_copy(src_hbm.at[pl.ds(srow, KSTR)], idx_v)
    for half in range(2):
      pltpu.sync_copy(tp_hbm.at[pl.ds(srow + half * (KSTR // 2), KSTR // 2)],
                      tp_v)
      cps = [
          pltpu.async_copy(tp_v.at[jj],
                           acc.at[idx_v.at[half * (KSTR // 2) + jj]], sem,
                           add=True)
          for jj in range(KSTR // 2)
      ]
      for cp in cps:
        cp.wait()
    return carry

  lax.fori_loop(0, nchunks, chunk, 0)
  plsc.subcore_barrier()
  pltpu.sync_copy(acc.at[pl.ds(s * rows_per_sub, rows_per_sub)],
                  out_hbm.at[c, pl.ds(s * rows_per_sub, rows_per_sub)])


def _sc_scatter(tp3d, src2d, zeros_n):
  nstreams = tp3d.shape[0]
  nchunks = nstreams // (NW * KSTR)
  n_pad = zeros_n.shape[0]
  mesh = plsc.VectorSubcoreMesh(
      core_axis_name="c", subcore_axis_name="s", num_cores=NC,
      num_subcores=NSUB)
  return pl.kernel(
      functools.partial(_sc_scatter_body, nchunks, n_pad),
      out_type=jax.ShapeDtypeStruct((NC, n_pad, TPW), jnp.float32),
      mesh=mesh,
      scratch_types=[
          pltpu.VMEM((KSTR, BATCH), jnp.int32),
          pltpu.VMEM((KSTR // 2, BATCH, TPW), jnp.float32),
          pltpu.VMEM_SHARED((n_pad, TPW), jnp.float32),
          pltpu.SemaphoreType.DMA,
      ],
      compiler_params=pltpu.CompilerParams(use_tc_tiling_on_sc=False),
  )(tp3d, src2d, zeros_n)


def _tc_final_body(p0_ref, p1_ref, out_ref):
  tot = p0_ref[...] + p1_ref[...]
  cnt = jnp.maximum(tot[:, 28:29], 1.0)
  out_ref[...] = tot[:, :28] / cnt


def _tc_final(p0, p1, n, block_n):
  return pl.pallas_call(
      _tc_final_body,
      grid=(n // block_n,),
      in_specs=[
          pl.BlockSpec((block_n, TPW), lambda i: (i, 0)),
          pl.BlockSpec((block_n, TPW), lambda i: (i, 0)),
      ],
      out_specs=pl.BlockSpec((block_n, 28), lambda i: (i, 0)),
      out_shape=jax.ShapeDtypeStruct((n, 28), jnp.float32),
      compiler_params=pltpu.CompilerParams(
          dimension_semantics=("arbitrary",)),
  )(p0, p1)


def _rearrange_w2(W_fc2, b_fc2):
  src = np.zeros((WPAD,), dtype=np.int32)
  g = NS + NV
  for i in range(NS):
    for j in range(NS):
      src[g * i + j] = NS * i + j
    for j in range(NV):
      src[g * i + NS + j] = W0N + NV * i + j
  w2p = W_fc2[:, src]
  b2p = b_fc2[src]
  return w2p, b2p


def _const_mats():
  rmat = np.zeros((NS, WPAD), dtype=np.float32)
  fmat = np.zeros((WPAD, 28), dtype=np.float32)
  mmat = np.zeros((NS, 28), dtype=np.float32)
  g = NS + NV
  for i in range(NS):
    for j in range(g):
      rmat[i, g * i + j] = 1.0
    for j in range(NS):
      fmat[g * i + j, j] = NORM
    for j in range(NV):
      for c in range(3):
        fmat[g * i + NS + j, NS + 3 * j + c] = NORM
  mmat[0, 0:NS] = 1.0
  for j in range(NV):
    for c in range(3):
      mmat[1 + c, NS + 3 * j + c] = 1.0
  return jnp.asarray(rmat), jnp.asarray(fmat), jnp.asarray(mmat)


def kernel(node_attr, edge_index, edge_attr, edge_sh, W_fc1, b_fc1, W_fc2,
           b_fc2):
  e = edge_attr.shape[0]
  n = node_attr.shape[0]
  src = edge_index[0].astype(jnp.int32)
  dst = edge_index[1].astype(jnp.int32)
  dst2d = dst.reshape(e // BATCH, BATCH)
  src2d = src.reshape(e // BATCH, BATCH)

  w2p, b2p = _rearrange_w2(W_fc2, b_fc2)
  b1_8 = jnp.broadcast_to(b_fc1[None, :], (8, NFEAT))
  b2p_8 = jnp.broadcast_to(b2p[None, :], (8, WPAD))

  rmat, fmat, mmat = _const_mats()
  x3d = _sc_gather(node_attr, dst2d)
  x = x3d.reshape(e, NS)
  tp = _tc_dense(x, edge_attr, edge_sh, W_fc1, b1_8, w2p, b2p_8,
                 rmat, fmat, mmat, block_e=8000)
  n_pad = ((n // NSUB + 7) // 8 * 8) * NSUB
  zeros_n = jnp.zeros((n_pad, TPW), jnp.float32)
  partials = _sc_scatter(tp.reshape(e // BATCH, BATCH, TPW), src2d, zeros_n)
  out = _tc_final(partials[0], partials[1], n, block_n=2000)
  return out
